# baseline (speedup 1.0000x reference)
import functools

import jax
import jax.numpy as jnp
from jax import lax
from jax.experimental import pallas as pl
from jax.experimental.pallas import tpu as pltpu
from jax.experimental.pallas import tpu_sc as plsc

NN = 10000
NP = 10240
EE = 320000
NC, NS = 2, 16
NT = NC * NS
CH = 128
NB = 8
NGRP = 10
NCHUNK = NB * NGRP
NCPAD = NCHUNK + NB
EPT = CH * NCHUNK
EPAD = NT * CH * NCPAD
FW = 16
RPT = NP // NS
NG = 64
BN = 2048
GRID = NP // BN

_mesh = plsc.VectorSubcoreMesh(
    core_axis_name="c", subcore_axis_name="s", num_cores=NC, num_subcores=NS
)

_sc_params = pltpu.CompilerParams(use_tc_tiling_on_sc=False)


@functools.partial(
    pl.kernel,
    out_type=jax.ShapeDtypeStruct((NC, NP, FW), jnp.float32),
    mesh=_mesh,
    scratch_types=[
        pltpu.VMEM((NCHUNK, CH), jnp.int32),
        pltpu.VMEM((CH, FW), jnp.float32),
        pltpu.VMEM((RPT, FW), jnp.float32),
        pltpu.VMEM_SHARED((NP, FW), jnp.float32),
        pltpu.SemaphoreType.DMA,
    ],
    compiler_params=_sc_params,
)
def _sc_degree(dst3, zeros_hbm, ones_hbm, out, idx_v, ones_v, buf_v, acc_sh,
               sem):
    cid = lax.axis_index("c")
    sid = lax.axis_index("s")
    wid = cid * NS + sid
    pltpu.sync_copy(dst3.at[wid, pl.ds(0, NCHUNK)], idx_v)
    pltpu.sync_copy(ones_hbm, ones_v)
    pltpu.sync_copy(zeros_hbm.at[pl.ds(sid * RPT, RPT)], buf_v)
    pltpu.sync_copy(buf_v, acc_sh.at[pl.ds(sid * RPT, RPT)])
    plsc.subcore_barrier()

    def body(j, carry):
        pltpu.async_copy(ones_v, acc_sh.at[idx_v.at[j]], sem, add=True)
        return carry

    lax.fori_loop(0, NCHUNK, body, 0)

    def drain(j, carry):
        pltpu.make_async_copy(ones_v, acc_sh.at[idx_v.at[j]], sem).wait()
        return carry

    lax.fori_loop(0, NCHUNK, drain, 0)
    plsc.subcore_barrier()
    pltpu.sync_copy(acc_sh.at[pl.ds(sid * RPT, RPT)], buf_v)
    pltpu.sync_copy(buf_v, out.at[cid, pl.ds(sid * RPT, RPT)])


@functools.partial(
    pl.kernel,
    out_type=jax.ShapeDtypeStruct((NC, NP, FW), jnp.float32),
    mesh=_mesh,
    scratch_types=[
        pltpu.VMEM((NCPAD, CH), jnp.int32),
        pltpu.VMEM((NCHUNK, CH), jnp.int32),
        pltpu.VMEM((2, NB, CH, FW), jnp.float32),
        pltpu.VMEM((RPT, FW), jnp.float32),
        pltpu.VMEM_SHARED((NP, FW), jnp.float32),
        pltpu.SemaphoreType.DMA,
        pltpu.SemaphoreType.DMA,
        pltpu.SemaphoreType.DMA,
        pltpu.SemaphoreType.DMA,
    ],
    compiler_params=_sc_params,
)
def _sc_aggregate(src3, dst3, table, zeros_hbm, out,
                  si_v, di_v, rows_v, buf_v, acc_sh,
                  sem_ga, sem_gb, sem_sa, sem_sb):
    cid = lax.axis_index("c")
    sid = lax.axis_index("s")
    wid = cid * NS + sid
    pltpu.sync_copy(src3.at[wid], si_v)
    pltpu.sync_copy(dst3.at[wid, pl.ds(0, NCHUNK)], di_v)
    pltpu.sync_copy(zeros_hbm.at[pl.ds(sid * RPT, RPT)], buf_v)
    pltpu.sync_copy(buf_v, acc_sh.at[pl.ds(sid * RPT, RPT)])
    plsc.subcore_barrier()

    def gathers(row0, bank, sem):
        for b in range(NB):
            pltpu.async_copy(
                table.at[si_v.at[row0 + b]], rows_v.at[bank, b], sem
            )

    def wait_gathers(row0, bank, sem):
        for b in range(NB):
            pltpu.make_async_copy(
                table.at[si_v.at[row0 + b]], rows_v.at[bank, b], sem
            ).wait()

    def scatters(row0, bank, sem):
        for b in range(NB):
            pltpu.async_copy(
                rows_v.at[bank, b], acc_sh.at[di_v.at[row0 + b]], sem,
                add=True,
            )
        for b in range(NB):
            pltpu.make_async_copy(
                rows_v.at[bank, b], acc_sh.at[di_v.at[row0 + b]], sem
            ).wait()

    gathers(0, 0, sem_ga)

    def body(g2, carry):
        r = g2 * 2 * NB
        gathers(r + NB, 1, sem_gb)
        wait_gathers(r, 0, sem_ga)
        scatters(r, 0, sem_sa)
        gathers(r + 2 * NB, 0, sem_ga)
        wait_gathers(r + NB, 1, sem_gb)
        scatters(r + NB, 1, sem_sb)
        return carry

    lax.fori_loop(0, NGRP // 2, body, 0)
    wait_gathers(NCHUNK, 0, sem_ga)
    plsc.subcore_barrier()
    pltpu.sync_copy(acc_sh.at[pl.ds(sid * RPT, RPT)], buf_v)
    pltpu.sync_copy(buf_v, out.at[cid, pl.ds(sid * RPT, RPT)])


def _tc1_body(x_ref, w1_ref, degp_ref, g1_ref, dinv_ref):
    d = degp_ref[...]
    deg = d[0] + d[1] + 1.0
    dinv = lax.rsqrt(deg)
    h = jnp.dot(x_ref[...], w1_ref[...], preferred_element_type=jnp.float32)
    g1_ref[...] = dinv * h
    dinv_ref[...] = dinv


def _tc2_body(p_ref, g1_ref, dinv_ref, b1_ref, w2_ref, g2_ref):
    p = p_ref[...]
    dinv = dinv_ref[...]
    t = dinv * (p[0] + p[1] + g1_ref[...]) + b1_ref[...]
    h1p = jnp.maximum(t, 0.0)
    g2_ref[...] = dinv * jnp.dot(
        h1p, w2_ref[...], preferred_element_type=jnp.float32
    )


def _tc3_body(p_ref, g2_ref, dinv_ref, b2_ref, batch_ref, out_ref, acc_ref):
    i = pl.program_id(0)

    @pl.when(i == 0)
    def _init():
        acc_ref[...] = jnp.zeros_like(acc_ref)

    p = p_ref[...]
    nodes = dinv_ref[...] * (p[0] + p[1] + g2_ref[...]) + b2_ref[...]
    col = lax.broadcasted_iota(jnp.int32, (BN, FW), 1)
    nodes = jnp.where(col < 10, nodes, jnp.where(col == 10, 1.0, 0.0))
    bvals = batch_ref[...].reshape(1, BN)
    gid = lax.broadcasted_iota(jnp.int32, (NG, BN), 0)
    mask = (gid == jnp.broadcast_to(bvals, (NG, BN))).astype(jnp.float32)
    acc_ref[...] += jnp.dot(mask, nodes, preferred_element_type=jnp.float32)

    @pl.when(i == GRID - 1)
    def _finish():
        a = acc_ref[...]
        cnt = jnp.maximum(a[:, 10:11], 1.0)
        v = a / cnt
        colv = lax.broadcasted_iota(jnp.int32, (NG, FW), 1)
        m = jnp.max(jnp.where(colv < 10, v, -1e30), axis=1, keepdims=True)
        e = jnp.where(colv < 10, jnp.exp(v - m), 0.0)
        lse = jnp.log(jnp.sum(e, axis=1, keepdims=True))
        out_ref[...] = (v - m - lse)[:, :10]


_tc1 = pl.pallas_call(
    _tc1_body,
    grid=(GRID,),
    in_specs=[
        pl.BlockSpec((BN, 128), lambda i: (i, 0)),
        pl.BlockSpec((128, FW), lambda i: (0, 0)),
        pl.BlockSpec((NC, BN, FW), lambda i: (0, i, 0)),
    ],
    out_specs=[
        pl.BlockSpec((BN, FW), lambda i: (i, 0)),
        pl.BlockSpec((BN, FW), lambda i: (i, 0)),
    ],
    out_shape=[
        jax.ShapeDtypeStruct((NP, FW), jnp.float32),
        jax.ShapeDtypeStruct((NP, FW), jnp.float32),
    ],
)

_tc2 = pl.pallas_call(
    _tc2_body,
    grid=(GRID,),
    in_specs=[
        pl.BlockSpec((NC, BN, FW), lambda i: (0, i, 0)),
        pl.BlockSpec((BN, FW), lambda i: (i, 0)),
        pl.BlockSpec((BN, FW), lambda i: (i, 0)),
        pl.BlockSpec((1, FW), lambda i: (0, 0)),
        pl.BlockSpec((FW, FW), lambda i: (0, 0)),
    ],
    out_specs=pl.BlockSpec((BN, FW), lambda i: (i, 0)),
    out_shape=jax.ShapeDtypeStruct((NP, FW), jnp.float32),
)

_tc3 = pl.pallas_call(
    _tc3_body,
    grid=(GRID,),
    in_specs=[
        pl.BlockSpec((NC, BN, FW), lambda i: (0, i, 0)),
        pl.BlockSpec((BN, FW), lambda i: (i, 0)),
        pl.BlockSpec((BN, FW), lambda i: (i, 0)),
        pl.BlockSpec((1, FW), lambda i: (0, 0)),
        pl.BlockSpec((1, 1, BN), lambda i: (i, 0, 0)),
    ],
    out_specs=pl.BlockSpec((NG, 10), lambda i: (0, 0)),
    out_shape=jax.ShapeDtypeStruct((NG, 10), jnp.float32),
    scratch_shapes=[pltpu.VMEM((NG, FW), jnp.float32)],
)


@jax.jit
def kernel(x, edge_index, batch, W1, b1, W2, b2):
    x = x.astype(jnp.float32)
    ei = edge_index.astype(jnp.int32)
    batch = batch.astype(jnp.int32)

    pad_e = jnp.full((NT * EPT - EE,), NN, jnp.int32)
    scrap_rows = jnp.full((NT, NCPAD - NCHUNK, CH), NN, jnp.int32)

    def to3(e):
        real = jnp.concatenate([e, pad_e]).reshape(NT, NCHUNK, CH)
        return jnp.concatenate([real, scrap_rows], axis=1)

    src3 = to3(ei[0])
    dst3 = to3(ei[1])
    xp = jnp.concatenate([x, jnp.zeros((NP - NN, x.shape[1]), jnp.float32)])
    zeros_tab = jnp.zeros((NP, FW), jnp.float32)
    ones_rows = jnp.ones((CH, FW), jnp.float32)
    batch3 = jnp.concatenate(
        [batch, jnp.full((NP - NN,), NG, jnp.int32)]
    ).reshape(GRID, 1, BN)
    b1r = b1.astype(jnp.float32).reshape(1, FW)
    w2p = jnp.pad(W2.astype(jnp.float32), ((0, 0), (0, FW - W2.shape[1])))
    b2r = jnp.pad(b2.astype(jnp.float32), (0, FW - b2.shape[0])).reshape(1, FW)

    degp = _sc_degree(dst3, zeros_tab, ones_rows)
    g1, dinv16 = _tc1(xp, W1.astype(jnp.float32), degp)
    p1 = _sc_aggregate(src3, dst3, g1, zeros_tab)
    g2 = _tc2(p1, g1, dinv16, b1r, w2p)
    p2 = _sc_aggregate(src3, dst3, g2, zeros_tab)
    return _tc3(p2, g2, dinv16, b2r, batch3)


# speedup vs baseline: 1.9442x; 1.9442x over previous
import functools

import jax
import jax.numpy as jnp
from jax import lax
from jax.experimental import pallas as pl
from jax.experimental.pallas import tpu as pltpu
from jax.experimental.pallas import tpu_sc as plsc

NN = 10000
NP = 10240
EE = 320000
NC, NS = 2, 16
NT = NC * NS
CH = 128
NCHUNK = 80
NSPLIT = 2
HCH = NCHUNK // NSPLIT
EPH = HCH * CH
EPT = CH * NCHUNK
EPAD = NT * EPT
FW = 16
RPT = NP // NS
NG = 64
BN = 2048
GRID = NP // BN

_mesh = plsc.VectorSubcoreMesh(
    core_axis_name="c", subcore_axis_name="s", num_cores=NC, num_subcores=NS
)

_sc_params = pltpu.CompilerParams(use_tc_tiling_on_sc=False)


@functools.partial(
    pl.kernel,
    out_type=jax.ShapeDtypeStruct((NC, NP, FW), jnp.float32),
    mesh=_mesh,
    scratch_types=[
        pltpu.VMEM((NSPLIT, EPH), jnp.int32),
        pltpu.VMEM((EPH, FW), jnp.float32),
        pltpu.VMEM((RPT, FW), jnp.float32),
        pltpu.VMEM_SHARED((NP, FW), jnp.float32),
        pltpu.SemaphoreType.DMA,
    ],
    compiler_params=_sc_params,
)
def _sc_degree(dst3, zeros_hbm, ones_hbm, out, idx_v, ones_v, buf_v, acc_sh,
               sem):
    cid = lax.axis_index("c")
    sid = lax.axis_index("s")
    wid = cid * NS + sid
    pltpu.sync_copy(dst3.at[wid], idx_v)
    pltpu.sync_copy(ones_hbm, ones_v)
    pltpu.sync_copy(zeros_hbm.at[pl.ds(sid * RPT, RPT)], buf_v)
    pltpu.sync_copy(buf_v, acc_sh.at[pl.ds(sid * RPT, RPT)])
    plsc.subcore_barrier()
    for h in range(NSPLIT):
        pltpu.async_copy(ones_v, acc_sh.at[idx_v.at[h]], sem, add=True)
    for h in range(NSPLIT):
        pltpu.make_async_copy(ones_v, acc_sh.at[idx_v.at[h]], sem).wait()
    plsc.subcore_barrier()
    pltpu.sync_copy(acc_sh.at[pl.ds(sid * RPT, RPT)], buf_v)
    pltpu.sync_copy(buf_v, out.at[cid, pl.ds(sid * RPT, RPT)])


@functools.partial(
    pl.kernel,
    out_type=jax.ShapeDtypeStruct((NC, NP, FW), jnp.float32),
    mesh=_mesh,
    scratch_types=[
        pltpu.VMEM((NSPLIT, EPH), jnp.int32),
        pltpu.VMEM((NSPLIT, EPH), jnp.int32),
        pltpu.VMEM((EPH, FW), jnp.float32),
        pltpu.VMEM((RPT, FW), jnp.float32),
        pltpu.VMEM_SHARED((NP, FW), jnp.float32),
        pltpu.SemaphoreType.DMA,
    ],
    compiler_params=_sc_params,
)
def _sc_aggregate(src3, dst3, table, zeros_hbm, out,
                  si_v, di_v, rows_v, buf_v, acc_sh, sem):
    cid = lax.axis_index("c")
    sid = lax.axis_index("s")
    wid = cid * NS + sid
    pltpu.sync_copy(src3.at[wid], si_v)
    pltpu.sync_copy(dst3.at[wid], di_v)
    pltpu.sync_copy(zeros_hbm.at[pl.ds(sid * RPT, RPT)], buf_v)
    pltpu.sync_copy(buf_v, acc_sh.at[pl.ds(sid * RPT, RPT)])
    plsc.subcore_barrier()

    for h in range(NSPLIT):
        pltpu.async_copy(table.at[si_v.at[h]], rows_v, sem).wait()
        pltpu.sync_copy(rows_v, acc_sh.at[di_v.at[h]], add=True)

    plsc.subcore_barrier()
    pltpu.sync_copy(acc_sh.at[pl.ds(sid * RPT, RPT)], buf_v)
    pltpu.sync_copy(buf_v, out.at[cid, pl.ds(sid * RPT, RPT)])


def _tc1_body(x_ref, w1_ref, degp_ref, g1_ref, dinv_ref):
    d = degp_ref[...]
    deg = d[0] + d[1] + 1.0
    dinv = lax.rsqrt(deg)
    h = jnp.dot(x_ref[...], w1_ref[...], preferred_element_type=jnp.float32)
    g1_ref[...] = dinv * h
    dinv_ref[...] = dinv


def _tc2_body(p_ref, g1_ref, dinv_ref, b1_ref, w2_ref, g2_ref):
    p = p_ref[...]
    dinv = dinv_ref[...]
    t = dinv * (p[0] + p[1] + g1_ref[...]) + b1_ref[...]
    h1p = jnp.maximum(t, 0.0)
    g2_ref[...] = dinv * jnp.dot(
        h1p, w2_ref[...], preferred_element_type=jnp.float32
    )


def _tc3_body(p_ref, g2_ref, dinv_ref, b2_ref, batch_ref, out_ref, acc_ref):
    i = pl.program_id(0)

    @pl.when(i == 0)
    def _init():
        acc_ref[...] = jnp.zeros_like(acc_ref)

    p = p_ref[...]
    nodes = dinv_ref[...] * (p[0] + p[1] + g2_ref[...]) + b2_ref[...]
    col = lax.broadcasted_iota(jnp.int32, (BN, FW), 1)
    nodes = jnp.where(col < 10, nodes, jnp.where(col == 10, 1.0, 0.0))
    bvals = batch_ref[...].reshape(1, BN)
    gid = lax.broadcasted_iota(jnp.int32, (NG, BN), 0)
    mask = (gid == jnp.broadcast_to(bvals, (NG, BN))).astype(jnp.float32)
    acc_ref[...] += jnp.dot(mask, nodes, preferred_element_type=jnp.float32)

    @pl.when(i == GRID - 1)
    def _finish():
        a = acc_ref[...]
        cnt = jnp.maximum(a[:, 10:11], 1.0)
        v = a / cnt
        colv = lax.broadcasted_iota(jnp.int32, (NG, FW), 1)
        m = jnp.max(jnp.where(colv < 10, v, -1e30), axis=1, keepdims=True)
        e = jnp.where(colv < 10, jnp.exp(v - m), 0.0)
        lse = jnp.log(jnp.sum(e, axis=1, keepdims=True))
        out_ref[...] = (v - m - lse)[:, :10]


_tc1 = pl.pallas_call(
    _tc1_body,
    grid=(GRID,),
    in_specs=[
        pl.BlockSpec((BN, 128), lambda i: (i, 0)),
        pl.BlockSpec((128, FW), lambda i: (0, 0)),
        pl.BlockSpec((NC, BN, FW), lambda i: (0, i, 0)),
    ],
    out_specs=[
        pl.BlockSpec((BN, FW), lambda i: (i, 0)),
        pl.BlockSpec((BN, FW), lambda i: (i, 0)),
    ],
    out_shape=[
        jax.ShapeDtypeStruct((NP, FW), jnp.float32),
        jax.ShapeDtypeStruct((NP, FW), jnp.float32),
    ],
)

_tc2 = pl.pallas_call(
    _tc2_body,
    grid=(GRID,),
    in_specs=[
        pl.BlockSpec((NC, BN, FW), lambda i: (0, i, 0)),
        pl.BlockSpec((BN, FW), lambda i: (i, 0)),
        pl.BlockSpec((BN, FW), lambda i: (i, 0)),
        pl.BlockSpec((1, FW), lambda i: (0, 0)),
        pl.BlockSpec((FW, FW), lambda i: (0, 0)),
    ],
    out_specs=pl.BlockSpec((BN, FW), lambda i: (i, 0)),
    out_shape=jax.ShapeDtypeStruct((NP, FW), jnp.float32),
)

_tc3 = pl.pallas_call(
    _tc3_body,
    grid=(GRID,),
    in_specs=[
        pl.BlockSpec((NC, BN, FW), lambda i: (0, i, 0)),
        pl.BlockSpec((BN, FW), lambda i: (i, 0)),
        pl.BlockSpec((BN, FW), lambda i: (i, 0)),
        pl.BlockSpec((1, FW), lambda i: (0, 0)),
        pl.BlockSpec((1, 1, BN), lambda i: (i, 0, 0)),
    ],
    out_specs=pl.BlockSpec((NG, 10), lambda i: (0, 0)),
    out_shape=jax.ShapeDtypeStruct((NG, 10), jnp.float32),
    scratch_shapes=[pltpu.VMEM((NG, FW), jnp.float32)],
)


@jax.jit
def kernel(x, edge_index, batch, W1, b1, W2, b2):
    x = x.astype(jnp.float32)
    ei = edge_index.astype(jnp.int32)
    batch = batch.astype(jnp.int32)

    pad_e = jnp.full((EPAD - EE,), NN, jnp.int32)
    src3 = jnp.concatenate([ei[0], pad_e]).reshape(NT, NSPLIT, EPH)
    dst3 = jnp.concatenate([ei[1], pad_e]).reshape(NT, NSPLIT, EPH)
    xp = jnp.concatenate([x, jnp.zeros((NP - NN, x.shape[1]), jnp.float32)])
    zeros_tab = jnp.zeros((NP, FW), jnp.float32)
    ones_rows = jnp.ones((EPH, FW), jnp.float32)
    batch3 = jnp.concatenate(
        [batch, jnp.full((NP - NN,), NG, jnp.int32)]
    ).reshape(GRID, 1, BN)
    b1r = b1.astype(jnp.float32).reshape(1, FW)
    w2p = jnp.pad(W2.astype(jnp.float32), ((0, 0), (0, FW - W2.shape[1])))
    b2r = jnp.pad(b2.astype(jnp.float32), (0, FW - b2.shape[0])).reshape(1, FW)

    degp = _sc_degree(dst3, zeros_tab, ones_rows)
    g1, dinv16 = _tc1(xp, W1.astype(jnp.float32), degp)
    p1 = _sc_aggregate(src3, dst3, g1, zeros_tab)
    g2 = _tc2(p1, g1, dinv16, b1r, w2p)
    p2 = _sc_aggregate(src3, dst3, g2, zeros_tab)
    return _tc3(p2, g2, dinv16, b2r, batch3)


# speedup vs baseline: 3.0346x; 1.5608x over previous
import functools

import jax
import jax.numpy as jnp
from jax import lax
from jax.experimental import pallas as pl
from jax.experimental.pallas import tpu as pltpu
from jax.experimental.pallas import tpu_sc as plsc

NN = 10000
NP = 10240
EE = 320000
NC, NS = 2, 16
NT = NC * NS
NSPLIT = 4
EPH = 2560
EPT = NSPLIT * EPH
EPAD = NT * EPT
DW = 8
FW = 16
RPT = NP // NS
NG = 64
BN = 2048
GRID = NP // BN

_mesh = plsc.VectorSubcoreMesh(
    core_axis_name="c", subcore_axis_name="s", num_cores=NC, num_subcores=NS
)

_sc_params = pltpu.CompilerParams(use_tc_tiling_on_sc=False)


@functools.partial(
    pl.kernel,
    out_type=jax.ShapeDtypeStruct((NC, NP, DW), jnp.float32),
    mesh=_mesh,
    scratch_types=[
        pltpu.VMEM((NSPLIT, EPH), jnp.int32),
        pltpu.VMEM((EPH, DW), jnp.float32),
        pltpu.VMEM_SHARED((NP, DW), jnp.float32),
        pltpu.SemaphoreType.DMA,
    ],
    compiler_params=_sc_params,
)
def _sc_degree(dst3, zeros_hbm, ones_hbm, out, idx_v, ones_v, acc_sh, sem):
    cid = lax.axis_index("c")
    sid = lax.axis_index("s")
    wid = cid * NS + sid
    pltpu.sync_copy(dst3.at[wid], idx_v)
    pltpu.sync_copy(ones_hbm, ones_v)
    pltpu.sync_copy(
        zeros_hbm.at[pl.ds(sid * RPT, RPT)], acc_sh.at[pl.ds(sid * RPT, RPT)]
    )
    plsc.subcore_barrier()
    descs = [
        pltpu.async_copy(ones_v, acc_sh.at[idx_v.at[h]], sem, add=True)
        for h in range(NSPLIT)
    ]
    for d in descs:
        d.wait()
    plsc.subcore_barrier()
    pltpu.sync_copy(
        acc_sh.at[pl.ds(sid * RPT, RPT)], out.at[cid, pl.ds(sid * RPT, RPT)]
    )


@functools.partial(
    pl.kernel,
    out_type=jax.ShapeDtypeStruct((NC, NP, FW), jnp.float32),
    mesh=_mesh,
    scratch_types=[
        pltpu.VMEM((NSPLIT, EPH), jnp.int32),
        pltpu.VMEM((NSPLIT, EPH), jnp.int32),
        pltpu.VMEM((2, EPH, FW), jnp.float32),
        pltpu.VMEM_SHARED((NP, FW), jnp.float32),
        pltpu.VMEM_SHARED((NP, FW), jnp.float32),
        pltpu.SemaphoreType.DMA,
        pltpu.SemaphoreType.DMA,
        pltpu.SemaphoreType.DMA,
        pltpu.SemaphoreType.DMA,
    ],
    compiler_params=_sc_params,
)
def _sc_aggregate(src3, dst3, table, zeros_hbm, out,
                  si_v, di_v, rows_v, tbl_sh, acc_sh,
                  sg0, sg1, ss0, ss1):
    cid = lax.axis_index("c")
    sid = lax.axis_index("s")
    wid = cid * NS + sid
    pltpu.sync_copy(src3.at[wid], si_v)
    pltpu.sync_copy(dst3.at[wid], di_v)
    pltpu.sync_copy(
        table.at[pl.ds(sid * RPT, RPT)], tbl_sh.at[pl.ds(sid * RPT, RPT)]
    )
    pltpu.sync_copy(
        zeros_hbm.at[pl.ds(sid * RPT, RPT)], acc_sh.at[pl.ds(sid * RPT, RPT)]
    )
    plsc.subcore_barrier()

    semg = (sg0, sg1)
    sems = (ss0, ss1)
    dg = [None] * NSPLIT
    dsc = [None] * NSPLIT
    dg[0] = pltpu.async_copy(tbl_sh.at[si_v.at[0]], rows_v.at[0], semg[0])
    for q in range(NSPLIT):
        bank = q % 2
        dg[q].wait()
        dsc[q] = pltpu.async_copy(
            rows_v.at[bank], acc_sh.at[di_v.at[q]], sems[bank], add=True
        )
        if q + 1 < NSPLIT:
            if q >= 1:
                dsc[q - 1].wait()
            dg[q + 1] = pltpu.async_copy(
                tbl_sh.at[si_v.at[q + 1]], rows_v.at[1 - bank],
                semg[1 - bank],
            )
    dsc[NSPLIT - 2].wait()
    dsc[NSPLIT - 1].wait()
    plsc.subcore_barrier()
    pltpu.sync_copy(
        acc_sh.at[pl.ds(sid * RPT, RPT)], out.at[cid, pl.ds(sid * RPT, RPT)]
    )


def _tc1_body(x_ref, w1_ref, degp_ref, g1_ref, dinv_ref):
    d = degp_ref[...]
    deg = (d[0] + d[1] + 1.0)[:, :1]
    dinv = jnp.broadcast_to(lax.rsqrt(deg), (BN, FW))
    h = jnp.dot(x_ref[...], w1_ref[...], preferred_element_type=jnp.float32)
    g1_ref[...] = dinv * h
    dinv_ref[...] = dinv


def _tc2_body(p_ref, g1_ref, dinv_ref, b1_ref, w2_ref, g2_ref):
    p = p_ref[...]
    dinv = dinv_ref[...]
    t = dinv * (p[0] + p[1] + g1_ref[...]) + b1_ref[...]
    h1p = jnp.maximum(t, 0.0)
    g2_ref[...] = dinv * jnp.dot(
        h1p, w2_ref[...], preferred_element_type=jnp.float32
    )


def _tc3_body(p_ref, g2_ref, dinv_ref, b2_ref, batch_ref, out_ref, acc_ref):
    i = pl.program_id(0)

    @pl.when(i == 0)
    def _init():
        acc_ref[...] = jnp.zeros_like(acc_ref)

    p = p_ref[...]
    nodes = dinv_ref[...] * (p[0] + p[1] + g2_ref[...]) + b2_ref[...]
    col = lax.broadcasted_iota(jnp.int32, (BN, FW), 1)
    nodes = jnp.where(col < 10, nodes, jnp.where(col == 10, 1.0, 0.0))
    bvals = batch_ref[...].reshape(1, BN)
    gid = lax.broadcasted_iota(jnp.int32, (NG, BN), 0)
    mask = (gid == jnp.broadcast_to(bvals, (NG, BN))).astype(jnp.float32)
    acc_ref[...] += jnp.dot(mask, nodes, preferred_element_type=jnp.float32)

    @pl.when(i == GRID - 1)
    def _finish():
        a = acc_ref[...]
        cnt = jnp.maximum(a[:, 10:11], 1.0)
        v = a / cnt
        colv = lax.broadcasted_iota(jnp.int32, (NG, FW), 1)
        m = jnp.max(jnp.where(colv < 10, v, -1e30), axis=1, keepdims=True)
        e = jnp.where(colv < 10, jnp.exp(v - m), 0.0)
        lse = jnp.log(jnp.sum(e, axis=1, keepdims=True))
        out_ref[...] = (v - m - lse)[:, :10]


_tc1 = pl.pallas_call(
    _tc1_body,
    grid=(GRID,),
    in_specs=[
        pl.BlockSpec((BN, 128), lambda i: (i, 0)),
        pl.BlockSpec((128, FW), lambda i: (0, 0)),
        pl.BlockSpec((NC, BN, DW), lambda i: (0, i, 0)),
    ],
    out_specs=[
        pl.BlockSpec((BN, FW), lambda i: (i, 0)),
        pl.BlockSpec((BN, FW), lambda i: (i, 0)),
    ],
    out_shape=[
        jax.ShapeDtypeStruct((NP, FW), jnp.float32),
        jax.ShapeDtypeStruct((NP, FW), jnp.float32),
    ],
)

_tc2 = pl.pallas_call(
    _tc2_body,
    grid=(GRID,),
    in_specs=[
        pl.BlockSpec((NC, BN, FW), lambda i: (0, i, 0)),
        pl.BlockSpec((BN, FW), lambda i: (i, 0)),
        pl.BlockSpec((BN, FW), lambda i: (i, 0)),
        pl.BlockSpec((1, FW), lambda i: (0, 0)),
        pl.BlockSpec((FW, FW), lambda i: (0, 0)),
    ],
    out_specs=pl.BlockSpec((BN, FW), lambda i: (i, 0)),
    out_shape=jax.ShapeDtypeStruct((NP, FW), jnp.float32),
)

_tc3 = pl.pallas_call(
    _tc3_body,
    grid=(GRID,),
    in_specs=[
        pl.BlockSpec((NC, BN, FW), lambda i: (0, i, 0)),
        pl.BlockSpec((BN, FW), lambda i: (i, 0)),
        pl.BlockSpec((BN, FW), lambda i: (i, 0)),
        pl.BlockSpec((1, FW), lambda i: (0, 0)),
        pl.BlockSpec((1, 1, BN), lambda i: (i, 0, 0)),
    ],
    out_specs=pl.BlockSpec((NG, 10), lambda i: (0, 0)),
    out_shape=jax.ShapeDtypeStruct((NG, 10), jnp.float32),
    scratch_shapes=[pltpu.VMEM((NG, FW), jnp.float32)],
)


@jax.jit
def kernel(x, edge_index, batch, W1, b1, W2, b2):
    x = x.astype(jnp.float32)
    ei = edge_index.astype(jnp.int32)
    batch = batch.astype(jnp.int32)

    pad_e = jnp.full((EPAD - EE,), NN, jnp.int32)
    src3 = jnp.concatenate([ei[0], pad_e]).reshape(NT, NSPLIT, EPH)
    dst3 = jnp.concatenate([ei[1], pad_e]).reshape(NT, NSPLIT, EPH)
    xp = jnp.concatenate([x, jnp.zeros((NP - NN, x.shape[1]), jnp.float32)])
    zeros_tab = jnp.zeros((NP, FW), jnp.float32)
    zeros_deg = jnp.zeros((NP, DW), jnp.float32)
    ones_rows = jnp.ones((EPH, DW), jnp.float32)
    batch3 = jnp.concatenate(
        [batch, jnp.full((NP - NN,), NG, jnp.int32)]
    ).reshape(GRID, 1, BN)
    b1r = b1.astype(jnp.float32).reshape(1, FW)
    w2p = jnp.pad(W2.astype(jnp.float32), ((0, 0), (0, FW - W2.shape[1])))
    b2r = jnp.pad(b2.astype(jnp.float32), (0, FW - b2.shape[0])).reshape(1, FW)

    degp = _sc_degree(dst3, zeros_deg, ones_rows)
    g1, dinv16 = _tc1(xp, W1.astype(jnp.float32), degp)
    p1 = _sc_aggregate(src3, dst3, g1, zeros_tab)
    g2 = _tc2(p1, g1, dinv16, b1r, w2p)
    p2 = _sc_aggregate(src3, dst3, g2, zeros_tab)
    return _tc3(p2, g2, dinv16, b2r, batch3)


# speedup vs baseline: 4.7438x; 1.5632x over previous
import functools

import jax
import jax.numpy as jnp
from jax import lax
from jax.experimental import pallas as pl
from jax.experimental.pallas import tpu as pltpu
from jax.experimental.pallas import tpu_sc as plsc

NN = 10000
NP = 10240
EE = 320000
NC, NS = 2, 16
NT = NC * NS
NSPLIT = 4
EPH = EE // NT // NSPLIT
FW = 16
RPT = NP // NS
NG = 64
BN = 2048
BNR = BN * FW // 128
NPP = NP * FW // 128
GRID = NP // BN

_mesh = plsc.VectorSubcoreMesh(
    core_axis_name="c", subcore_axis_name="s", num_cores=NC, num_subcores=NS
)

_sc_params = pltpu.CompilerParams(use_tc_tiling_on_sc=False)


@functools.partial(
    pl.kernel,
    out_type=jax.ShapeDtypeStruct((NC, NP, FW), jnp.float32),
    mesh=_mesh,
    scratch_types=[
        pltpu.VMEM((NSPLIT, EPH), jnp.int32),
        pltpu.VMEM((EPH, FW), jnp.float32),
        pltpu.VMEM_SHARED((NP, FW), jnp.float32),
        pltpu.SemaphoreType.DMA,
    ],
    compiler_params=_sc_params,
)
def _sc_degree(dst3, zeros_hbm, ones_hbm, out, idx_v, ones_v, acc_sh, sem):
    cid = lax.axis_index("c")
    sid = lax.axis_index("s")
    wid = cid * NS + sid
    pltpu.sync_copy(dst3.at[wid], idx_v)
    pltpu.sync_copy(ones_hbm, ones_v)
    pltpu.sync_copy(
        zeros_hbm.at[pl.ds(sid * RPT, RPT)], acc_sh.at[pl.ds(sid * RPT, RPT)]
    )
    plsc.subcore_barrier()
    descs = [
        pltpu.async_copy(ones_v, acc_sh.at[idx_v.at[h]], sem, add=True)
        for h in range(NSPLIT)
    ]
    for d in descs:
        d.wait()
    plsc.subcore_barrier()
    pltpu.sync_copy(
        acc_sh.at[pl.ds(sid * RPT, RPT)], out.at[cid, pl.ds(sid * RPT, RPT)]
    )


@functools.partial(
    pl.kernel,
    out_type=jax.ShapeDtypeStruct((NC, NP, FW), jnp.float32),
    mesh=_mesh,
    scratch_types=[
        pltpu.VMEM((NSPLIT, EPH), jnp.int32),
        pltpu.VMEM((NSPLIT, EPH), jnp.int32),
        pltpu.VMEM((2, EPH, FW), jnp.float32),
        pltpu.VMEM_SHARED((NP, FW), jnp.float32),
        pltpu.VMEM_SHARED((NP, FW), jnp.float32),
        pltpu.SemaphoreType.DMA,
        pltpu.SemaphoreType.DMA,
        pltpu.SemaphoreType.DMA,
        pltpu.SemaphoreType.DMA,
    ],
    compiler_params=_sc_params,
)
def _sc_aggregate(src3, dst3, table, zeros_hbm, out,
                  si_v, di_v, rows_v, tbl_sh, acc_sh,
                  sg0, sg1, ss0, ss1):
    cid = lax.axis_index("c")
    sid = lax.axis_index("s")
    wid = cid * NS + sid
    pltpu.sync_copy(src3.at[wid], si_v)
    pltpu.sync_copy(dst3.at[wid], di_v)
    pltpu.sync_copy(
        table.at[pl.ds(sid * RPT, RPT)], tbl_sh.at[pl.ds(sid * RPT, RPT)]
    )
    pltpu.sync_copy(
        zeros_hbm.at[pl.ds(sid * RPT, RPT)], acc_sh.at[pl.ds(sid * RPT, RPT)]
    )
    plsc.subcore_barrier()

    semg = (sg0, sg1)
    sems = (ss0, ss1)
    dg = [None] * NSPLIT
    dsc = [None] * NSPLIT
    dg[0] = pltpu.async_copy(tbl_sh.at[si_v.at[0]], rows_v.at[0], semg[0])
    for q in range(NSPLIT):
        bank = q % 2
        dg[q].wait()
        dsc[q] = pltpu.async_copy(
            rows_v.at[bank], acc_sh.at[di_v.at[q]], sems[bank], add=True
        )
        if q + 1 < NSPLIT:
            if q >= 1:
                dsc[q - 1].wait()
            dg[q + 1] = pltpu.async_copy(
                tbl_sh.at[si_v.at[q + 1]], rows_v.at[1 - bank],
                semg[1 - bank],
            )
    dsc[NSPLIT - 2].wait()
    dsc[NSPLIT - 1].wait()
    plsc.subcore_barrier()
    pltpu.sync_copy(
        acc_sh.at[pl.ds(sid * RPT, RPT)], out.at[cid, pl.ds(sid * RPT, RPT)]
    )


def _tc1_body(x_ref, w1k_ref, degp_ref, g1_ref, dinv_ref):
    d = degp_ref[...]
    dinv = lax.rsqrt(d[0] + d[1] + 1.0)
    hp = jnp.dot(x_ref[...], w1k_ref[...], preferred_element_type=jnp.float32)
    g1_ref[...] = dinv * hp
    dinv_ref[...] = dinv


def _tc2_body(p_ref, g1_ref, dinv_ref, b1t_ref, w2k_ref, g2_ref):
    p = p_ref[...]
    dinv = dinv_ref[...]
    t = dinv * (p[0] + p[1] + g1_ref[...]) + b1t_ref[...]
    h1p = jnp.maximum(t, 0.0)
    g2_ref[...] = dinv * jnp.dot(
        h1p, w2k_ref[...], preferred_element_type=jnp.float32
    )


def _tc3_body(p_ref, g2_ref, dinv_ref, b2_ref, batch_ref, out_ref, acc_ref):
    i = pl.program_id(0)

    @pl.when(i == 0)
    def _init():
        acc_ref[...] = jnp.zeros_like(acc_ref)

    p = p_ref[...]
    nodes_p = dinv_ref[...] * (p[0] + p[1] + g2_ref[...])
    col16 = lax.broadcasted_iota(jnp.int32, (BNR, 128), 1) % FW
    nodes_p = jnp.where(
        col16 < 10, nodes_p, jnp.where(col16 == 10, 1.0, 0.0)
    )
    b3 = batch_ref[...].reshape(8, BNR)
    gid = lax.broadcasted_iota(jnp.int32, (NG, BNR), 0)
    total = jnp.zeros((NG, FW), jnp.float32)
    for k in range(8):
        row = b3[k:k + 1]
        mask = (gid == jnp.broadcast_to(row, (NG, BNR))).astype(jnp.float32)
        mm = jnp.dot(mask, nodes_p, preferred_element_type=jnp.float32)
        total += mm[:, FW * k:FW * (k + 1)]
    acc_ref[...] += total

    @pl.when(i == GRID - 1)
    def _finish():
        a = acc_ref[...]
        cnt = a[:, 10:11]
        v = a / jnp.maximum(cnt, 1.0) + b2_ref[...] * (cnt > 0.0)
        colv = lax.broadcasted_iota(jnp.int32, (NG, FW), 1)
        m = jnp.max(jnp.where(colv < 10, v, -1e30), axis=1, keepdims=True)
        e = jnp.where(colv < 10, jnp.exp(v - m), 0.0)
        lse = jnp.log(jnp.sum(e, axis=1, keepdims=True))
        out_ref[...] = (v - m - lse)[:, :10]


_tc1 = pl.pallas_call(
    _tc1_body,
    grid=(GRID,),
    in_specs=[
        pl.BlockSpec((BNR, 1024), lambda i: (i, 0)),
        pl.BlockSpec((1024, 128), lambda i: (0, 0)),
        pl.BlockSpec((NC, BNR, 128), lambda i: (0, i, 0)),
    ],
    out_specs=[
        pl.BlockSpec((BNR, 128), lambda i: (i, 0)),
        pl.BlockSpec((BNR, 128), lambda i: (i, 0)),
    ],
    out_shape=[
        jax.ShapeDtypeStruct((NPP, 128), jnp.float32),
        jax.ShapeDtypeStruct((NPP, 128), jnp.float32),
    ],
)

_tc2 = pl.pallas_call(
    _tc2_body,
    grid=(GRID,),
    in_specs=[
        pl.BlockSpec((NC, BNR, 128), lambda i: (0, i, 0)),
        pl.BlockSpec((BNR, 128), lambda i: (i, 0)),
        pl.BlockSpec((BNR, 128), lambda i: (i, 0)),
        pl.BlockSpec((1, 128), lambda i: (0, 0)),
        pl.BlockSpec((128, 128), lambda i: (0, 0)),
    ],
    out_specs=pl.BlockSpec((BNR, 128), lambda i: (i, 0)),
    out_shape=jax.ShapeDtypeStruct((NPP, 128), jnp.float32),
)

_tc3 = pl.pallas_call(
    _tc3_body,
    grid=(GRID,),
    in_specs=[
        pl.BlockSpec((NC, BNR, 128), lambda i: (0, i, 0)),
        pl.BlockSpec((BNR, 128), lambda i: (i, 0)),
        pl.BlockSpec((BNR, 128), lambda i: (i, 0)),
        pl.BlockSpec((1, FW), lambda i: (0, 0)),
        pl.BlockSpec((1, 8, BNR), lambda i: (i, 0, 0)),
    ],
    out_specs=pl.BlockSpec((NG, 10), lambda i: (0, 0)),
    out_shape=jax.ShapeDtypeStruct((NG, 10), jnp.float32),
    scratch_shapes=[pltpu.VMEM((NG, FW), jnp.float32)],
)


@jax.jit
def kernel(x, edge_index, batch, W1, b1, W2, b2):
    x = x.astype(jnp.float32)
    ei = edge_index.astype(jnp.int32)
    batch = batch.astype(jnp.int32)

    src3 = ei[0].reshape(NT, NSPLIT, EPH)
    dst3 = ei[1].reshape(NT, NSPLIT, EPH)
    xp = jnp.concatenate([x, jnp.zeros((NP - NN, x.shape[1]), jnp.float32)])
    xp8 = xp.reshape(NPP, 1024)
    zeros_tab = jnp.zeros((NP, FW), jnp.float32)
    ones_rows = jnp.ones((EPH, FW), jnp.float32)
    batchk = (
        jnp.concatenate([batch, jnp.full((NP - NN,), NG, jnp.int32)])
        .reshape(GRID, BNR, 8)
        .transpose(0, 2, 1)
    )
    eye8 = jnp.eye(8, dtype=jnp.float32)
    w1k = jnp.kron(eye8, W1.astype(jnp.float32))
    w2p = jnp.pad(W2.astype(jnp.float32), ((0, 0), (0, FW - W2.shape[1])))
    w2k = jnp.kron(eye8, w2p)
    b1t = jnp.tile(b1.astype(jnp.float32), 8).reshape(1, 128)
    b2r = jnp.pad(b2.astype(jnp.float32), (0, FW - b2.shape[0])).reshape(1, FW)

    degp = _sc_degree(dst3, zeros_tab, ones_rows)
    g1p, dinvp = _tc1(xp8, w1k, degp.reshape(NC, NPP, 128))
    p1 = _sc_aggregate(src3, dst3, g1p.reshape(NP, FW), zeros_tab)
    g2p = _tc2(p1.reshape(NC, NPP, 128), g1p, dinvp, b1t, w2k)
    p2 = _sc_aggregate(src3, dst3, g2p.reshape(NP, FW), zeros_tab)
    return _tc3(p2.reshape(NC, NPP, 128), g2p, dinvp, b2r, batchk)


# speedup vs baseline: 5.0195x; 1.0581x over previous
import functools

import jax
import jax.numpy as jnp
from jax import lax
from jax.experimental import pallas as pl
from jax.experimental.pallas import tpu as pltpu
from jax.experimental.pallas import tpu_sc as plsc

NN = 10000
NP = 10240
EE = 320000
NC, NS = 2, 16
NT = NC * NS
NSPLIT = 4
EPH = EE // NT // NSPLIT
FW = 16
RPT = NP // NS
NG = 64
BN = 2048
BNR = BN * FW // 128
NPP = NP * FW // 128
GRID = NP // BN

_mesh = plsc.VectorSubcoreMesh(
    core_axis_name="c", subcore_axis_name="s", num_cores=NC, num_subcores=NS
)

_sc_params = pltpu.CompilerParams(use_tc_tiling_on_sc=False)


@functools.partial(
    pl.kernel,
    out_type=jax.ShapeDtypeStruct((NC, NP, FW), jnp.float32),
    mesh=_mesh,
    scratch_types=[
        pltpu.VMEM((NSPLIT, EPH), jnp.int32),
        pltpu.VMEM((EPH, FW), jnp.float32),
        pltpu.VMEM_SHARED((NP, FW), jnp.float32),
        pltpu.SemaphoreType.DMA,
    ],
    compiler_params=_sc_params,
)
def _sc_degree(ei4, zeros_hbm, ones_hbm, out, idx_v, ones_v, acc_sh, sem):
    cid = lax.axis_index("c")
    sid = lax.axis_index("s")
    wid = cid * NS + sid
    pltpu.sync_copy(ei4.at[1, wid], idx_v)
    pltpu.sync_copy(ones_hbm, ones_v)
    pltpu.sync_copy(
        zeros_hbm.at[pl.ds(sid * RPT, RPT)], acc_sh.at[pl.ds(sid * RPT, RPT)]
    )
    plsc.subcore_barrier()
    descs = [
        pltpu.async_copy(ones_v, acc_sh.at[idx_v.at[h]], sem, add=True)
        for h in range(NSPLIT)
    ]
    for d in descs:
        d.wait()
    plsc.subcore_barrier()
    pltpu.sync_copy(
        acc_sh.at[pl.ds(sid * RPT, RPT)], out.at[cid, pl.ds(sid * RPT, RPT)]
    )


@functools.partial(
    pl.kernel,
    out_type=jax.ShapeDtypeStruct((NC, NP, FW), jnp.float32),
    mesh=_mesh,
    scratch_types=[
        pltpu.VMEM((NSPLIT, EPH), jnp.int32),
        pltpu.VMEM((NSPLIT, EPH), jnp.int32),
        pltpu.VMEM((2, EPH, FW), jnp.float32),
        pltpu.VMEM_SHARED((NP, FW), jnp.float32),
        pltpu.VMEM_SHARED((NP, FW), jnp.float32),
        pltpu.SemaphoreType.DMA,
        pltpu.SemaphoreType.DMA,
        pltpu.SemaphoreType.DMA,
        pltpu.SemaphoreType.DMA,
    ],
    compiler_params=_sc_params,
)
def _sc_aggregate(ei4, table, zeros_hbm, out,
                  si_v, di_v, rows_v, tbl_sh, acc_sh,
                  sg0, sg1, ss0, ss1):
    cid = lax.axis_index("c")
    sid = lax.axis_index("s")
    wid = cid * NS + sid
    pltpu.sync_copy(ei4.at[0, wid], si_v)
    pltpu.sync_copy(ei4.at[1, wid], di_v)
    pltpu.sync_copy(
        table.at[pl.ds(sid * RPT, RPT)], tbl_sh.at[pl.ds(sid * RPT, RPT)]
    )
    pltpu.sync_copy(
        zeros_hbm.at[pl.ds(sid * RPT, RPT)], acc_sh.at[pl.ds(sid * RPT, RPT)]
    )
    plsc.subcore_barrier()

    semg = (sg0, sg1)
    sems = (ss0, ss1)
    dg = [None] * NSPLIT
    dsc = [None] * NSPLIT
    dg[0] = pltpu.async_copy(tbl_sh.at[si_v.at[0]], rows_v.at[0], semg[0])
    for q in range(NSPLIT):
        bank = q % 2
        dg[q].wait()
        dsc[q] = pltpu.async_copy(
            rows_v.at[bank], acc_sh.at[di_v.at[q]], sems[bank], add=True
        )
        if q + 1 < NSPLIT:
            if q >= 1:
                dsc[q - 1].wait()
            dg[q + 1] = pltpu.async_copy(
                tbl_sh.at[si_v.at[q + 1]], rows_v.at[1 - bank],
                semg[1 - bank],
            )
    dsc[NSPLIT - 2].wait()
    dsc[NSPLIT - 1].wait()
    plsc.subcore_barrier()
    pltpu.sync_copy(
        acc_sh.at[pl.ds(sid * RPT, RPT)], out.at[cid, pl.ds(sid * RPT, RPT)]
    )


def _tc1_body(x_ref, w1k_ref, degp_ref, g1_ref, dinv_ref):
    d = degp_ref[...]
    dinv = lax.rsqrt(d[0] + d[1] + 1.0)
    hp = jnp.dot(x_ref[...], w1k_ref[...], preferred_element_type=jnp.float32)
    g1_ref[...] = dinv * hp
    dinv_ref[...] = dinv


def _tc2_body(p_ref, g1_ref, dinv_ref, b1t_ref, w2k_ref, g2_ref):
    p = p_ref[...]
    dinv = dinv_ref[...]
    t = dinv * (p[0] + p[1] + g1_ref[...]) + b1t_ref[...]
    h1p = jnp.maximum(t, 0.0)
    g2_ref[...] = dinv * jnp.dot(
        h1p, w2k_ref[...], preferred_element_type=jnp.float32
    )


def _tc3_body(p_ref, g2_ref, dinv_ref, b2_ref, batch_ref, out_ref, acc_ref):
    i = pl.program_id(0)

    @pl.when(i == 0)
    def _init():
        acc_ref[...] = jnp.zeros_like(acc_ref)

    p = p_ref[...]
    nodes_p = dinv_ref[...] * (p[0] + p[1] + g2_ref[...])
    col16 = lax.broadcasted_iota(jnp.int32, (BNR, 128), 1) % FW
    nodes_p = jnp.where(
        col16 < 10, nodes_p, jnp.where(col16 == 10, 1.0, 0.0)
    )
    b3 = batch_ref[...].reshape(8, BNR)
    gid = lax.broadcasted_iota(jnp.int32, (NG, BNR), 0)
    total = jnp.zeros((NG, FW), jnp.float32)
    for k in range(8):
        row = b3[k:k + 1]
        mask = (gid == jnp.broadcast_to(row, (NG, BNR))).astype(jnp.float32)
        mm = jnp.dot(mask, nodes_p, preferred_element_type=jnp.float32)
        total += mm[:, FW * k:FW * (k + 1)]
    acc_ref[...] += total

    @pl.when(i == GRID - 1)
    def _finish():
        a = acc_ref[...]
        cnt = a[:, 10:11]
        v = a / jnp.maximum(cnt, 1.0) + b2_ref[...] * (cnt > 0.0)
        colv = lax.broadcasted_iota(jnp.int32, (NG, FW), 1)
        m = jnp.max(jnp.where(colv < 10, v, -1e30), axis=1, keepdims=True)
        e = jnp.where(colv < 10, jnp.exp(v - m), 0.0)
        lse = jnp.log(jnp.sum(e, axis=1, keepdims=True))
        out_ref[...] = (v - m - lse)[:, :10]


_tc1 = pl.pallas_call(
    _tc1_body,
    grid=(GRID,),
    in_specs=[
        pl.BlockSpec((BNR, 1024), lambda i: (i, 0)),
        pl.BlockSpec((1024, 128), lambda i: (0, 0)),
        pl.BlockSpec((NC, BNR, 128), lambda i: (0, i, 0)),
    ],
    out_specs=[
        pl.BlockSpec((BNR, 128), lambda i: (i, 0)),
        pl.BlockSpec((BNR, 128), lambda i: (i, 0)),
    ],
    out_shape=[
        jax.ShapeDtypeStruct((NPP, 128), jnp.float32),
        jax.ShapeDtypeStruct((NPP, 128), jnp.float32),
    ],
)

_tc2 = pl.pallas_call(
    _tc2_body,
    grid=(GRID,),
    in_specs=[
        pl.BlockSpec((NC, BNR, 128), lambda i: (0, i, 0)),
        pl.BlockSpec((BNR, 128), lambda i: (i, 0)),
        pl.BlockSpec((BNR, 128), lambda i: (i, 0)),
        pl.BlockSpec((1, 128), lambda i: (0, 0)),
        pl.BlockSpec((128, 128), lambda i: (0, 0)),
    ],
    out_specs=pl.BlockSpec((BNR, 128), lambda i: (i, 0)),
    out_shape=jax.ShapeDtypeStruct((NPP, 128), jnp.float32),
)

_tc3 = pl.pallas_call(
    _tc3_body,
    grid=(GRID,),
    in_specs=[
        pl.BlockSpec((NC, BNR, 128), lambda i: (0, i, 0)),
        pl.BlockSpec((BNR, 128), lambda i: (i, 0)),
        pl.BlockSpec((BNR, 128), lambda i: (i, 0)),
        pl.BlockSpec((1, FW), lambda i: (0, 0)),
        pl.BlockSpec((1, 8, BNR), lambda i: (i, 0, 0)),
    ],
    out_specs=pl.BlockSpec((NG, 10), lambda i: (0, 0)),
    out_shape=jax.ShapeDtypeStruct((NG, 10), jnp.float32),
    scratch_shapes=[pltpu.VMEM((NG, FW), jnp.float32)],
)


@jax.jit
def kernel(x, edge_index, batch, W1, b1, W2, b2):
    x = x.astype(jnp.float32)
    ei = edge_index.astype(jnp.int32)
    batch = batch.astype(jnp.int32)

    ei4 = ei.reshape(2, NT, NSPLIT, EPH)
    xp = jnp.concatenate([x, jnp.zeros((NP - NN, x.shape[1]), jnp.float32)])
    xp8 = xp.reshape(NPP, 1024)
    zeros_tab = jnp.zeros((NP, FW), jnp.float32)
    ones_rows = jnp.ones((EPH, FW), jnp.float32)
    batchk = (
        jnp.concatenate([batch, jnp.full((NP - NN,), NG, jnp.int32)])
        .reshape(GRID, BNR, 8)
        .transpose(0, 2, 1)
    )
    eye8 = jnp.eye(8, dtype=jnp.float32)
    w1k = jnp.kron(eye8, W1.astype(jnp.float32))
    w2p = jnp.pad(W2.astype(jnp.float32), ((0, 0), (0, FW - W2.shape[1])))
    w2k = jnp.kron(eye8, w2p)
    b1t = jnp.tile(b1.astype(jnp.float32), 8).reshape(1, 128)
    b2r = jnp.pad(b2.astype(jnp.float32), (0, FW - b2.shape[0])).reshape(1, FW)

    degp = _sc_degree(ei4, zeros_tab, ones_rows)
    g1p, dinvp = _tc1(xp8, w1k, degp.reshape(NC, NPP, 128))
    p1 = _sc_aggregate(ei4, g1p.reshape(NP, FW), zeros_tab)
    g2p = _tc2(p1.reshape(NC, NPP, 128), g1p, dinvp, b1t, w2k)
    p2 = _sc_aggregate(ei4, g2p.reshape(NP, FW), zeros_tab)
    return _tc3(p2.reshape(NC, NPP, 128), g2p, dinvp, b2r, batchk)


# speedup vs baseline: 5.0325x; 1.0026x over previous
import functools

import jax
import jax.numpy as jnp
import numpy as np
from jax import lax
from jax.experimental import pallas as pl
from jax.experimental.pallas import tpu as pltpu
from jax.experimental.pallas import tpu_sc as plsc

NN = 10000
NP = 10240
EE = 320000
NC, NS = 2, 16
NT = NC * NS
NSPLIT = 4
EPH = EE // NT // NSPLIT
FW = 16
RPT = NP // NS
NG = 64
BN = 2048
BNR = BN * FW // 128
NPP = NP * FW // 128
GRID = NP // BN

_mesh = plsc.VectorSubcoreMesh(
    core_axis_name="c", subcore_axis_name="s", num_cores=NC, num_subcores=NS
)

_sc_params = pltpu.CompilerParams(use_tc_tiling_on_sc=False)


@functools.partial(
    pl.kernel,
    out_type=jax.ShapeDtypeStruct((NC, NP, FW), jnp.float32),
    mesh=_mesh,
    scratch_types=[
        pltpu.VMEM((NSPLIT, EPH), jnp.int32),
        pltpu.VMEM((EPH, FW), jnp.float32),
        pltpu.VMEM_SHARED((NP, FW), jnp.float32),
        pltpu.SemaphoreType.DMA,
    ],
    compiler_params=_sc_params,
)
def _sc_degree(ei4, zeros_hbm, ones_hbm, out, idx_v, ones_v, acc_sh, sem):
    cid = lax.axis_index("c")
    sid = lax.axis_index("s")
    wid = cid * NS + sid
    pltpu.sync_copy(ei4.at[1, wid], idx_v)
    pltpu.sync_copy(ones_hbm, ones_v)
    pltpu.sync_copy(
        zeros_hbm.at[pl.ds(sid * RPT, RPT)], acc_sh.at[pl.ds(sid * RPT, RPT)]
    )
    plsc.subcore_barrier()
    descs = [
        pltpu.async_copy(ones_v, acc_sh.at[idx_v.at[h]], sem, add=True)
        for h in range(NSPLIT)
    ]
    for d in descs:
        d.wait()
    plsc.subcore_barrier()
    pltpu.sync_copy(
        acc_sh.at[pl.ds(sid * RPT, RPT)], out.at[cid, pl.ds(sid * RPT, RPT)]
    )


@functools.partial(
    pl.kernel,
    out_type=jax.ShapeDtypeStruct((NC, NP, FW), jnp.float32),
    mesh=_mesh,
    scratch_types=[
        pltpu.VMEM((NSPLIT, EPH), jnp.int32),
        pltpu.VMEM((NSPLIT, EPH), jnp.int32),
        pltpu.VMEM((2, EPH, FW), jnp.float32),
        pltpu.VMEM_SHARED((NP, FW), jnp.float32),
        pltpu.VMEM_SHARED((NP, FW), jnp.float32),
        pltpu.SemaphoreType.DMA,
        pltpu.SemaphoreType.DMA,
        pltpu.SemaphoreType.DMA,
        pltpu.SemaphoreType.DMA,
    ],
    compiler_params=_sc_params,
)
def _sc_aggregate(ei4, table, zeros_hbm, out,
                  si_v, di_v, rows_v, tbl_sh, acc_sh,
                  sg0, sg1, ss0, ss1):
    cid = lax.axis_index("c")
    sid = lax.axis_index("s")
    wid = cid * NS + sid
    pltpu.sync_copy(ei4.at[0, wid], si_v)
    pltpu.sync_copy(ei4.at[1, wid], di_v)
    pltpu.sync_copy(
        table.at[pl.ds(sid * RPT, RPT)], tbl_sh.at[pl.ds(sid * RPT, RPT)]
    )
    pltpu.sync_copy(
        zeros_hbm.at[pl.ds(sid * RPT, RPT)], acc_sh.at[pl.ds(sid * RPT, RPT)]
    )
    plsc.subcore_barrier()

    semg = (sg0, sg1)
    sems = (ss0, ss1)
    dg = [None] * NSPLIT
    dsc = [None] * NSPLIT
    dg[0] = pltpu.async_copy(tbl_sh.at[si_v.at[0]], rows_v.at[0], semg[0])
    for q in range(NSPLIT):
        bank = q % 2
        dg[q].wait()
        dsc[q] = pltpu.async_copy(
            rows_v.at[bank], acc_sh.at[di_v.at[q]], sems[bank], add=True
        )
        if q + 1 < NSPLIT:
            if q >= 1:
                dsc[q - 1].wait()
            dg[q + 1] = pltpu.async_copy(
                tbl_sh.at[si_v.at[q + 1]], rows_v.at[1 - bank],
                semg[1 - bank],
            )
    dsc[NSPLIT - 2].wait()
    dsc[NSPLIT - 1].wait()
    plsc.subcore_barrier()
    pltpu.sync_copy(
        acc_sh.at[pl.ds(sid * RPT, RPT)], out.at[cid, pl.ds(sid * RPT, RPT)]
    )


def _tc0_body(x_ref, w1k_ref, h1_ref):
    h1_ref[...] = jnp.dot(
        x_ref[...], w1k_ref[...], preferred_element_type=jnp.float32
    )


def _tc1_body(h1_ref, degp_ref, g1_ref, dinv_ref):
    d = degp_ref[...]
    dinv = lax.rsqrt(d[0] + d[1] + 1.0)
    g1_ref[...] = dinv * h1_ref[...]
    dinv_ref[...] = dinv


def _tc2_body(p_ref, g1_ref, dinv_ref, b1t_ref, w2k_ref, g2_ref):
    p = p_ref[...]
    dinv = dinv_ref[...]
    t = dinv * (p[0] + p[1] + g1_ref[...]) + b1t_ref[...]
    h1p = jnp.maximum(t, 0.0)
    g2_ref[...] = dinv * jnp.dot(
        h1p, w2k_ref[...], preferred_element_type=jnp.float32
    )


def _tc3_body(p_ref, g2_ref, dinv_ref, b2_ref, batch_ref, out_ref, acc_ref):
    i = pl.program_id(0)

    @pl.when(i == 0)
    def _init():
        acc_ref[...] = jnp.zeros_like(acc_ref)

    p = p_ref[...]
    nodes_p = dinv_ref[...] * (p[0] + p[1] + g2_ref[...])
    col16 = lax.broadcasted_iota(jnp.int32, (BNR, 128), 1) % FW
    nodes_p = jnp.where(
        col16 < 10, nodes_p, jnp.where(col16 == 10, 1.0, 0.0)
    )
    b3 = batch_ref[...].reshape(8, BNR)
    gid = lax.broadcasted_iota(jnp.int32, (NG, BNR), 0)
    total = jnp.zeros((NG, FW), jnp.float32)
    for k in range(8):
        row = b3[k:k + 1]
        mask = (gid == jnp.broadcast_to(row, (NG, BNR))).astype(jnp.float32)
        mm = jnp.dot(mask, nodes_p, preferred_element_type=jnp.float32)
        total += mm[:, FW * k:FW * (k + 1)]
    acc_ref[...] += total

    @pl.when(i == GRID - 1)
    def _finish():
        a = acc_ref[...]
        cnt = a[:, 10:11]
        v = a / jnp.maximum(cnt, 1.0) + b2_ref[...] * (cnt > 0.0)
        colv = lax.broadcasted_iota(jnp.int32, (NG, FW), 1)
        m = jnp.max(jnp.where(colv < 10, v, -1e30), axis=1, keepdims=True)
        e = jnp.where(colv < 10, jnp.exp(v - m), 0.0)
        lse = jnp.log(jnp.sum(e, axis=1, keepdims=True))
        out_ref[...] = (v - m - lse)[:, :10]


_tc0 = pl.pallas_call(
    _tc0_body,
    grid=(GRID,),
    in_specs=[
        pl.BlockSpec((BNR, 1024), lambda i: (i, 0)),
        pl.BlockSpec((1024, 128), lambda i: (0, 0)),
    ],
    out_specs=pl.BlockSpec((BNR, 128), lambda i: (i, 0)),
    out_shape=jax.ShapeDtypeStruct((NPP, 128), jnp.float32),
)

_tc1 = pl.pallas_call(
    _tc1_body,
    grid=(GRID,),
    in_specs=[
        pl.BlockSpec((BNR, 128), lambda i: (i, 0)),
        pl.BlockSpec((NC, BNR, 128), lambda i: (0, i, 0)),
    ],
    out_specs=[
        pl.BlockSpec((BNR, 128), lambda i: (i, 0)),
        pl.BlockSpec((BNR, 128), lambda i: (i, 0)),
    ],
    out_shape=[
        jax.ShapeDtypeStruct((NPP, 128), jnp.float32),
        jax.ShapeDtypeStruct((NPP, 128), jnp.float32),
    ],
)

_tc2 = pl.pallas_call(
    _tc2_body,
    grid=(GRID,),
    in_specs=[
        pl.BlockSpec((NC, BNR, 128), lambda i: (0, i, 0)),
        pl.BlockSpec((BNR, 128), lambda i: (i, 0)),
        pl.BlockSpec((BNR, 128), lambda i: (i, 0)),
        pl.BlockSpec((1, 128), lambda i: (0, 0)),
        pl.BlockSpec((128, 128), lambda i: (0, 0)),
    ],
    out_specs=pl.BlockSpec((BNR, 128), lambda i: (i, 0)),
    out_shape=jax.ShapeDtypeStruct((NPP, 128), jnp.float32),
)

_tc3 = pl.pallas_call(
    _tc3_body,
    grid=(GRID,),
    in_specs=[
        pl.BlockSpec((NC, BNR, 128), lambda i: (0, i, 0)),
        pl.BlockSpec((BNR, 128), lambda i: (i, 0)),
        pl.BlockSpec((BNR, 128), lambda i: (i, 0)),
        pl.BlockSpec((1, FW), lambda i: (0, 0)),
        pl.BlockSpec((1, 8, BNR), lambda i: (i, 0, 0)),
    ],
    out_specs=pl.BlockSpec((NG, 10), lambda i: (0, 0)),
    out_shape=jax.ShapeDtypeStruct((NG, 10), jnp.float32),
    scratch_shapes=[pltpu.VMEM((NG, FW), jnp.float32)],
)


@jax.jit
def kernel(x, edge_index, batch, W1, b1, W2, b2):
    x = x.astype(jnp.float32)
    ei = edge_index.astype(jnp.int32)
    batch = batch.astype(jnp.int32)

    ei4 = ei.reshape(2, NT, NSPLIT, EPH)
    xp = jnp.concatenate([x, jnp.zeros((NP - NN, x.shape[1]), jnp.float32)])
    xp8 = xp.reshape(NPP, 1024)
    zeros_tab = np.zeros((NP, FW), np.float32)
    ones_rows = np.ones((EPH, FW), np.float32)
    batchk = (
        jnp.concatenate([batch, jnp.full((NP - NN,), NG, jnp.int32)])
        .reshape(GRID, BNR, 8)
        .transpose(0, 2, 1)
    )
    eye8 = jnp.eye(8, dtype=jnp.float32)
    w1k = jnp.kron(eye8, W1.astype(jnp.float32))
    w2p = jnp.pad(W2.astype(jnp.float32), ((0, 0), (0, FW - W2.shape[1])))
    w2k = jnp.kron(eye8, w2p)
    b1t = jnp.tile(b1.astype(jnp.float32), 8).reshape(1, 128)
    b2r = jnp.pad(b2.astype(jnp.float32), (0, FW - b2.shape[0])).reshape(1, FW)

    degp = _sc_degree(ei4, zeros_tab, ones_rows)
    h1p = _tc0(xp8, w1k)
    g1p, dinvp = _tc1(h1p, degp.reshape(NC, NPP, 128))
    p1 = _sc_aggregate(ei4, g1p.reshape(NP, FW), zeros_tab)
    g2p = _tc2(p1.reshape(NC, NPP, 128), g1p, dinvp, b1t, w2k)
    p2 = _sc_aggregate(ei4, g2p.reshape(NP, FW), zeros_tab)
    return _tc3(p2.reshape(NC, NPP, 128), g2p, dinvp, b2r, batchk)


# speedup vs baseline: 5.0474x; 1.0030x over previous
import functools

import jax
import jax.numpy as jnp
import numpy as np
from jax import lax
from jax.experimental import pallas as pl
from jax.experimental.pallas import tpu as pltpu
from jax.experimental.pallas import tpu_sc as plsc

NN = 10000
NP = 10240
EE = 320000
NC, NS = 2, 16
NT = NC * NS
NSPLIT = 4
EPH = EE // NT // NSPLIT
FW = 16
RPT = NP // NS
NG = 64
BN = 2048
BNR = BN * FW // 128
NPP = NP * FW // 128
GRID = NP // BN

_mesh = plsc.VectorSubcoreMesh(
    core_axis_name="c", subcore_axis_name="s", num_cores=NC, num_subcores=NS
)

_sc_params = pltpu.CompilerParams(use_tc_tiling_on_sc=False)


@functools.partial(
    pl.kernel,
    out_type=jax.ShapeDtypeStruct((NC, NP, FW), jnp.float32),
    mesh=_mesh,
    scratch_types=[
        pltpu.VMEM((NSPLIT, EPH), jnp.int32),
        pltpu.VMEM((EPH, FW), jnp.float32),
        pltpu.VMEM_SHARED((NP, FW), jnp.float32),
        pltpu.SemaphoreType.DMA,
    ],
    compiler_params=_sc_params,
)
def _sc_degree(ei4, zeros_hbm, ones_hbm, out, idx_v, ones_v, acc_sh, sem):
    cid = lax.axis_index("c")
    sid = lax.axis_index("s")
    wid = cid * NS + sid
    pltpu.sync_copy(ei4.at[1, wid], idx_v)
    pltpu.sync_copy(ones_hbm, ones_v)
    pltpu.sync_copy(
        zeros_hbm.at[pl.ds(sid * RPT, RPT)], acc_sh.at[pl.ds(sid * RPT, RPT)]
    )
    plsc.subcore_barrier()
    descs = [
        pltpu.async_copy(ones_v, acc_sh.at[idx_v.at[h]], sem, add=True)
        for h in range(NSPLIT)
    ]
    for d in descs:
        d.wait()
    plsc.subcore_barrier()
    pltpu.sync_copy(
        acc_sh.at[pl.ds(sid * RPT, RPT)], out.at[cid, pl.ds(sid * RPT, RPT)]
    )


@functools.partial(
    pl.kernel,
    out_type=jax.ShapeDtypeStruct((NC, NP, FW), jnp.float32),
    mesh=_mesh,
    scratch_types=[
        pltpu.VMEM((NSPLIT, EPH), jnp.int32),
        pltpu.VMEM((NSPLIT, EPH), jnp.int32),
        pltpu.VMEM((2, EPH, FW), jnp.float32),
        pltpu.VMEM_SHARED((NP, FW), jnp.float32),
        pltpu.VMEM_SHARED((NP, FW), jnp.float32),
        pltpu.SemaphoreType.DMA,
        pltpu.SemaphoreType.DMA,
        pltpu.SemaphoreType.DMA,
        pltpu.SemaphoreType.DMA,
    ],
    compiler_params=_sc_params,
)
def _sc_aggregate(ei4, table, zeros_hbm, out,
                  si_v, di_v, rows_v, tbl_sh, acc_sh,
                  sg0, sg1, ss0, ss1):
    cid = lax.axis_index("c")
    sid = lax.axis_index("s")
    wid = cid * NS + sid
    pltpu.sync_copy(ei4.at[0, wid], si_v)
    pltpu.sync_copy(ei4.at[1, wid], di_v)
    pltpu.sync_copy(
        table.at[pl.ds(sid * RPT, RPT)], tbl_sh.at[pl.ds(sid * RPT, RPT)]
    )
    pltpu.sync_copy(
        zeros_hbm.at[pl.ds(sid * RPT, RPT)], acc_sh.at[pl.ds(sid * RPT, RPT)]
    )
    plsc.subcore_barrier()

    semg = (sg0, sg1)
    sems = (ss0, ss1)
    dg = [None] * NSPLIT
    dsc = [None] * NSPLIT
    dg[0] = pltpu.async_copy(tbl_sh.at[si_v.at[0]], rows_v.at[0], semg[0])
    for q in range(NSPLIT):
        bank = q % 2
        dg[q].wait()
        dsc[q] = pltpu.async_copy(
            rows_v.at[bank], acc_sh.at[di_v.at[q]], sems[bank], add=True
        )
        if q + 1 < NSPLIT:
            if q >= 1:
                dsc[q - 1].wait()
            dg[q + 1] = pltpu.async_copy(
                tbl_sh.at[si_v.at[q + 1]], rows_v.at[1 - bank],
                semg[1 - bank],
            )
    dsc[NSPLIT - 2].wait()
    dsc[NSPLIT - 1].wait()
    plsc.subcore_barrier()
    pltpu.sync_copy(
        acc_sh.at[pl.ds(sid * RPT, RPT)], out.at[cid, pl.ds(sid * RPT, RPT)]
    )


def _tc0_body(x_ref, w1k_ref, h1_ref):
    h1_ref[...] = jnp.dot(
        x_ref[...], w1k_ref[...], preferred_element_type=jnp.float32
    )


def _tc1_body(h1_ref, degp_ref, g1_ref, dinv_ref):
    d = degp_ref[...]
    dinv = lax.rsqrt(d[0] + d[1] + 1.0)
    g1_ref[...] = dinv * h1_ref[...]
    dinv_ref[...] = dinv


def _tc2_body(p_ref, g1_ref, dinv_ref, b1t_ref, w2k_ref, g2_ref):
    p = p_ref[...]
    dinv = dinv_ref[...]
    t = dinv * (p[0] + p[1] + g1_ref[...]) + b1t_ref[...]
    h1p = jnp.maximum(t, 0.0)
    g2_ref[...] = dinv * jnp.dot(
        h1p, w2k_ref[...], preferred_element_type=jnp.float32
    )


def _tc3_body(p_ref, g2_ref, dinv_ref, b2_ref, batch_ref, out_ref, acc_ref):
    i = pl.program_id(0)

    @pl.when(i == 0)
    def _init():
        acc_ref[...] = jnp.zeros_like(acc_ref)

    p = p_ref[...]
    nodes_p = dinv_ref[...] * (p[0] + p[1] + g2_ref[...])
    col16 = lax.broadcasted_iota(jnp.int32, (BNR, 128), 1) % FW
    nodes_p = jnp.where(
        col16 < 10, nodes_p, jnp.where(col16 == 10, 1.0, 0.0)
    )
    b3 = batch_ref[...].reshape(8, BNR)
    gid = lax.broadcasted_iota(jnp.int32, (NG, BNR), 0)
    total = jnp.zeros((NG, FW), jnp.float32)
    for k in range(8):
        row = b3[k:k + 1]
        mask = (gid == jnp.broadcast_to(row, (NG, BNR))).astype(jnp.float32)
        mm = jnp.dot(mask, nodes_p, preferred_element_type=jnp.float32)
        total += mm[:, FW * k:FW * (k + 1)]
    acc_ref[...] += total

    @pl.when(i == GRID - 1)
    def _finish():
        a = acc_ref[...]
        cnt = a[:, 10:11]
        v = a / jnp.maximum(cnt, 1.0) + b2_ref[...] * (cnt > 0.0)
        colv = lax.broadcasted_iota(jnp.int32, (NG, FW), 1)
        m = jnp.max(jnp.where(colv < 10, v, -1e30), axis=1, keepdims=True)
        e = jnp.where(colv < 10, jnp.exp(v - m), 0.0)
        lse = jnp.log(jnp.sum(e, axis=1, keepdims=True))
        out_ref[...] = (v - m - lse)[:, :10]


_tc0 = pl.pallas_call(
    _tc0_body,
    grid=(GRID,),
    in_specs=[
        pl.BlockSpec((BNR, 1024), lambda i: (i, 0)),
        pl.BlockSpec((1024, 128), lambda i: (0, 0)),
    ],
    out_specs=pl.BlockSpec((BNR, 128), lambda i: (i, 0)),
    out_shape=jax.ShapeDtypeStruct((NPP, 128), jnp.float32),
)

_tc1 = pl.pallas_call(
    _tc1_body,
    grid=(GRID,),
    in_specs=[
        pl.BlockSpec((BNR, 128), lambda i: (i, 0)),
        pl.BlockSpec((NC, BNR, 128), lambda i: (0, i, 0)),
    ],
    out_specs=[
        pl.BlockSpec((BNR, 128), lambda i: (i, 0)),
        pl.BlockSpec((BNR, 128), lambda i: (i, 0)),
    ],
    out_shape=[
        jax.ShapeDtypeStruct((NPP, 128), jnp.float32),
        jax.ShapeDtypeStruct((NPP, 128), jnp.float32),
    ],
)

_tc2 = pl.pallas_call(
    _tc2_body,
    grid=(GRID,),
    in_specs=[
        pl.BlockSpec((NC, BNR, 128), lambda i: (0, i, 0)),
        pl.BlockSpec((BNR, 128), lambda i: (i, 0)),
        pl.BlockSpec((BNR, 128), lambda i: (i, 0)),
        pl.BlockSpec((1, 128), lambda i: (0, 0)),
        pl.BlockSpec((128, 128), lambda i: (0, 0)),
    ],
    out_specs=pl.BlockSpec((BNR, 128), lambda i: (i, 0)),
    out_shape=jax.ShapeDtypeStruct((NPP, 128), jnp.float32),
)

_tc3 = pl.pallas_call(
    _tc3_body,
    grid=(GRID,),
    in_specs=[
        pl.BlockSpec((NC, BNR, 128), lambda i: (0, i, 0)),
        pl.BlockSpec((BNR, 128), lambda i: (i, 0)),
        pl.BlockSpec((BNR, 128), lambda i: (i, 0)),
        pl.BlockSpec((1, FW), lambda i: (0, 0)),
        pl.BlockSpec((1, 8, BNR), lambda i: (i, 0, 0)),
    ],
    out_specs=pl.BlockSpec((NG, 10), lambda i: (0, 0)),
    out_shape=jax.ShapeDtypeStruct((NG, 10), jnp.float32),
    scratch_shapes=[pltpu.VMEM((NG, FW), jnp.float32)],
)


@jax.jit
def kernel(x, edge_index, batch, W1, b1, W2, b2):
    x = x.astype(jnp.float32)
    ei = edge_index.astype(jnp.int32)
    batch = batch.astype(jnp.int32)

    ei4 = ei.reshape(2, NT, NSPLIT, EPH)
    zeros_tab = jnp.zeros((NP, FW), jnp.float32)
    ones_rows = jnp.ones((EPH, FW), jnp.float32)
    degp = _sc_degree(ei4, zeros_tab, ones_rows)

    xp = jnp.concatenate([x, jnp.zeros((NP - NN, x.shape[1]), jnp.float32)])
    xp8 = xp.reshape(NPP, 1024)
    batchk = (
        jnp.concatenate([batch, jnp.full((NP - NN,), NG, jnp.int32)])
        .reshape(GRID, BNR, 8)
        .transpose(0, 2, 1)
    )
    eye8 = jnp.eye(8, dtype=jnp.float32)
    w1k = jnp.kron(eye8, W1.astype(jnp.float32))
    w2p = jnp.pad(W2.astype(jnp.float32), ((0, 0), (0, FW - W2.shape[1])))
    w2k = jnp.kron(eye8, w2p)
    b1t = jnp.tile(b1.astype(jnp.float32), 8).reshape(1, 128)
    b2r = jnp.pad(b2.astype(jnp.float32), (0, FW - b2.shape[0])).reshape(1, FW)

    h1p = _tc0(xp8, w1k)
    g1p, dinvp = _tc1(h1p, degp.reshape(NC, NPP, 128))
    p1 = _sc_aggregate(ei4, g1p.reshape(NP, FW), zeros_tab)
    g2p = _tc2(p1.reshape(NC, NPP, 128), g1p, dinvp, b1t, w2k)
    p2 = _sc_aggregate(ei4, g2p.reshape(NP, FW), zeros_tab)
    return _tc3(p2.reshape(NC, NPP, 128), g2p, dinvp, b2r, batchk)
